# baseline (speedup 1.0000x reference)
import functools
import jax
import jax.numpy as jnp
from jax import lax
from jax.experimental import pallas as pl
from jax.experimental.pallas import tpu as pltpu
from jax.experimental.pallas import tpu_sc as plsc

_B, _N, _MO, _D = 128, 32, 48, 64
_R = _N * _B
_NB = 4
_L = 16
_G = _B // _L


def _sc_desc_body(cats_hbm, subs_hbm, mask_hbm, cl_hbm, cr_hbm,
                  ca_hbm, va_hbm, cb_hbm, vb_hbm, cnt_hbm,
                  cats_v, subs_v, mask_v, cl_v, cr_v, vc_v,
                  vc_pub, ca_o, va_o, cb_o, vb_o, cnt_o, vc_sh):
    c = lax.axis_index("c")
    s = lax.axis_index("s")

    pltpu.sync_copy(cats_hbm, cats_v)
    pltpu.sync_copy(subs_hbm, subs_v)
    pltpu.sync_copy(mask_hbm, mask_v)
    pltpu.sync_copy(cl_hbm, cl_v)
    pltpu.sync_copy(cr_hbm, cr_v)

    lane = lax.iota(jnp.int32, _L)

    for r in range(2):
        row = 2 * s + r
        for g in range(_G):
            sl = pl.ds(row * _B + g * _L, _L)
            b_idx = lane + g * _L
            cats_g = cats_v[sl]
            subs_g = subs_v[sl]
            mask_g = mask_v[sl]
            cl_g = jnp.clip(cl_v[sl], 0, _N - 1)
            flat_l = cl_g * _B + b_idx
            cats_l = plsc.load_gather(cats_v, [flat_l])
            subs_l = plsc.load_gather(subs_v, [flat_l])
            mask_l = plsc.load_gather(mask_v, [flat_l])
            ecat = jnp.where(mask_g != 0, cats_g, 3)
            ecat_l = jnp.where(mask_l != 0, cats_l, 3)
            vpm = jnp.where(ecat == 0, subs_g + 1, subs_l + 1)
            cpm = jnp.where(ecat == 0, 1,
                            jnp.where((ecat == 1) & (ecat_l == 0), subs_g + 2, 0))
            vc_pub[pl.ds(r * _B + g * _L, _L)] = vpm + 8 * cpm

    pltpu.sync_copy(vc_pub, vc_sh.at[pl.ds(2 * s * _B, 2 * _B)])
    plsc.subcore_barrier()
    pltpu.sync_copy(vc_sh, vc_v)

    w = 2 * s + c
    for g in range(_G):
        sl = pl.ds(w * _B + g * _L, _L)
        osl = pl.ds(g * _L, _L)
        b_idx = lane + g * _L
        cats_g = cats_v[sl]
        subs_g = subs_v[sl]
        mask_g = mask_v[sl]
        cl_g = jnp.clip(cl_v[sl], 0, _N - 1)
        cr_g = jnp.clip(cr_v[sl], 0, _N - 1)
        vc_g = vc_v[sl]
        vpm = vc_g & 7
        cpm = vc_g >> 3
        ecat = jnp.where(mask_g != 0, cats_g, 3)
        is_after = subs_g == 1
        i_first = jnp.where(is_after, cr_g, cl_g)
        i_second = jnp.where(is_after, cl_g, cr_g)
        pk_f = plsc.load_gather(vc_v, [i_first * _B + b_idx])
        pk_s = plsc.load_gather(vc_v, [i_second * _B + b_idx])
        is_comb = ecat == 2
        c_a = jnp.where(is_comb, pk_f >> 3, cpm)
        v_a = jnp.where(is_comb, pk_f & 7, vpm)
        c_b = jnp.where(is_comb, pk_s >> 3, 0)
        v_b = pk_s & 7
        ca_o[osl] = c_a
        va_o[osl] = v_a
        cb_o[osl] = c_b
        vb_o[osl] = v_b
        cnt_o[osl] = (c_a + c_b).astype(jnp.float32)

    row_out = pl.ds(w * _B, _B)
    pltpu.sync_copy(ca_o, ca_hbm.at[row_out])
    pltpu.sync_copy(va_o, va_hbm.at[row_out])
    pltpu.sync_copy(cb_o, cb_hbm.at[row_out])
    pltpu.sync_copy(vb_o, vb_hbm.at[row_out])
    pltpu.sync_copy(cnt_o, cnt_hbm.at[row_out])


_sc_desc = functools.partial(
    pl.kernel,
    out_type=[jax.ShapeDtypeStruct((_R,), jnp.int32)] * 4
    + [jax.ShapeDtypeStruct((_R,), jnp.float32)],
    mesh=plsc.VectorSubcoreMesh(core_axis_name="c", subcore_axis_name="s"),
    compiler_params=pltpu.CompilerParams(needs_layout_passes=False),
    scratch_types=[pltpu.VMEM((_R,), jnp.int32)] * 6
    + [pltpu.VMEM((2 * _B,), jnp.int32)]
    + [pltpu.VMEM((_B,), jnp.int32)] * 4
    + [pltpu.VMEM((_B,), jnp.float32),
       pltpu.VMEM_SHARED((_R,), jnp.int32)],
)(_sc_desc_body)


def _expand_body(ca_ref, va_ref, cb_ref, vb_ref, e1_ref, e2_ref, out_ref):
    c_a = ca_ref[...]
    v_a = va_ref[...]
    c_b = cb_ref[...]
    v_b = vb_ref[...]
    e1 = e1_ref[...]
    e2 = e2_ref[...]
    zero = jnp.zeros((1, 1, 1, 1), jnp.float32)

    ea = jnp.where(v_a == 1, e1, jnp.where(v_a == 2, e2, zero))
    eb = jnp.where(v_b == 1, e1, jnp.where(v_b == 2, e2, zero))
    p = jax.lax.broadcasted_iota(jnp.int32, (1, _MO, 1, 1), 1)
    in_a = p < c_a
    in_ab = p < (c_a + c_b)
    out_ref[...] = jnp.where(in_a, ea, jnp.where(in_ab, eb, zero))


def kernel(node_cats, node_subs, node_mask, child_left, child_right, action_embed):
    mask_i = node_mask.astype(jnp.int32)
    c_a, v_a, c_b, v_b, cnt = _sc_desc(
        node_cats.T.reshape(_R), node_subs.T.reshape(_R), mask_i.T.reshape(_R),
        child_left.T.reshape(_R), child_right.T.reshape(_R))

    desc_spec = pl.BlockSpec((_NB, 1, 1, _B), lambda i: (i, 0, 0, 0))
    evec_spec = pl.BlockSpec((1, 1, _D, 1), lambda i: (0, 0, 0, 0))
    out = pl.pallas_call(
        _expand_body,
        grid=(_N // _NB,),
        in_specs=[desc_spec] * 4 + [evec_spec] * 2,
        out_specs=pl.BlockSpec((_NB, _MO, _D, _B), lambda i: (i, 0, 0, 0)),
        out_shape=jax.ShapeDtypeStruct((_N, _MO, _D, _B), jnp.float32),
    )(c_a.reshape(_N, 1, 1, _B), v_a.reshape(_N, 1, 1, _B),
      c_b.reshape(_N, 1, 1, _B), v_b.reshape(_N, 1, 1, _B),
      action_embed[1].reshape(1, 1, _D, 1), action_embed[2].reshape(1, 1, _D, 1))

    return jnp.transpose(out, (3, 0, 1, 2)), cnt.reshape(_N, _B).T


# speedup vs baseline: 1.8894x; 1.8894x over previous
import jax
import jax.numpy as jnp
from jax.experimental import pallas as pl

_B, _N, _MO, _D = 128, 32, 48, 64
_NB = 2


def _loop_gather(x, idx):
    acc = jnp.zeros_like(x)
    for j in range(_N):
        acc = jnp.where(idx == j, x[j:j + 1, :], acc)
    return acc


def _desc_body(cats_ref, subs_ref, mask_ref, cl_ref, cr_ref,
               ca_ref, va_ref, cb_ref, vb_ref, cnt_ref):
    cats = cats_ref[...]
    subs = subs_ref[...]
    msk = mask_ref[...]
    cl = jnp.clip(cl_ref[...], 0, _N - 1)
    cr = jnp.clip(cr_ref[...], 0, _N - 1)

    ecat = jnp.where(msk != 0, cats, 3)
    pk_l = _loop_gather(ecat + 4 * subs, cl)
    ecat_l = pk_l & 3
    subs_l = pk_l >> 2

    vpm = jnp.where(ecat == 0, subs + 1, subs_l + 1)
    cpm = jnp.where(ecat == 0, 1,
                    jnp.where((ecat == 1) & (ecat_l == 0), subs + 2, 0))

    is_after = subs == 1
    i_first = jnp.where(is_after, cr, cl)
    i_second = jnp.where(is_after, cl, cr)
    vc = vpm + 8 * cpm
    pk_f = _loop_gather(vc, i_first)
    pk_s = _loop_gather(vc, i_second)

    is_comb = ecat == 2
    c_a = jnp.where(is_comb, pk_f >> 3, cpm)
    v_a = jnp.where(is_comb, pk_f & 7, vpm)
    c_b = jnp.where(is_comb, pk_s >> 3, 0)
    v_b = pk_s & 7

    ca_ref[...] = c_a
    va_ref[...] = v_a
    cb_ref[...] = c_b
    vb_ref[...] = v_b
    cnt_ref[...] = (c_a + c_b).astype(jnp.float32)


def _expand_body(ca_ref, va_ref, cb_ref, vb_ref, e1_ref, e2_ref, out_ref):
    c_a = ca_ref[...]
    v_a = va_ref[...]
    c_b = cb_ref[...]
    v_b = vb_ref[...]
    e1 = e1_ref[...]
    e2 = e2_ref[...]
    zero = jnp.zeros((1, 1, 1, 1), jnp.float32)

    ea = jnp.where(v_a == 1, e1, jnp.where(v_a == 2, e2, zero))
    eb = jnp.where(v_b == 1, e1, jnp.where(v_b == 2, e2, zero))
    p = jax.lax.broadcasted_iota(jnp.int32, (1, _MO, 1, 1), 1)
    in_a = p < c_a
    in_ab = p < (c_a + c_b)
    out_ref[...] = jnp.where(in_a, ea, jnp.where(in_ab, eb, zero))


def kernel(node_cats, node_subs, node_mask, child_left, child_right, action_embed):
    mask_i = node_mask.astype(jnp.int32)
    col_spec = pl.BlockSpec((_N, _B), lambda: (0, 0))
    c_a, v_a, c_b, v_b, cnt_t = pl.pallas_call(
        _desc_body,
        in_specs=[col_spec] * 5,
        out_specs=[col_spec] * 5,
        out_shape=[jax.ShapeDtypeStruct((_N, _B), jnp.int32)] * 4
        + [jax.ShapeDtypeStruct((_N, _B), jnp.float32)],
    )(node_cats.T, node_subs.T, mask_i.T, child_left.T, child_right.T)

    desc_spec = pl.BlockSpec((_NB, 1, 1, _B), lambda i: (i, 0, 0, 0))
    evec_spec = pl.BlockSpec((1, 1, _D, 1), lambda i: (0, 0, 0, 0))
    out = pl.pallas_call(
        _expand_body,
        grid=(_N // _NB,),
        in_specs=[desc_spec] * 4 + [evec_spec] * 2,
        out_specs=pl.BlockSpec((_NB, _MO, _D, _B), lambda i: (i, 0, 0, 0)),
        out_shape=jax.ShapeDtypeStruct((_N, _MO, _D, _B), jnp.float32),
    )(c_a.reshape(_N, 1, 1, _B), v_a.reshape(_N, 1, 1, _B),
      c_b.reshape(_N, 1, 1, _B), v_b.reshape(_N, 1, 1, _B),
      action_embed[1].reshape(1, 1, _D, 1), action_embed[2].reshape(1, 1, _D, 1))

    return jnp.transpose(out, (3, 0, 1, 2)), cnt_t.T
